# baseline (speedup 1.0000x reference)
import functools

import jax
import jax.numpy as jnp
from jax.experimental import pallas as pl
from jax.experimental.pallas import tpu as pltpu

B, N, C = 64, 8732, 21
TOTAL = B * N
ROWS = 4096
GRID = (TOTAL + ROWS - 1) // ROWS

_ALPHA = 0.25
_OBJ_THRESH = 0.4


def _body(conf_ref, tgt_ref, osp_ref, ost_ref, loc1_ref, loc2_ref, loct_ref,
          out_ref):
    pid = pl.program_id(0)

    @pl.when(pid == 0)
    def _init():
        for i in range(8):
            out_ref[i] = 0.0

    row0 = pid * ROWS
    rows = jax.lax.broadcasted_iota(jnp.int32, (ROWS, 1), 0) + row0
    maskb = rows < TOTAL
    maskf = maskb.astype(jnp.float32)

    conf = jnp.where(maskb, conf_ref[...], 0.0)
    osp = jnp.where(maskb, osp_ref[...], 0.0)
    tgt = jnp.where(maskb, tgt_ref[...], 0)
    ost = jnp.where(maskb, ost_ref[...], 0)
    d1 = jnp.where(maskb, loc1_ref[...] - loct_ref[...], 0.0)
    d2 = jnp.where(maskb, loc2_ref[...] - loct_ref[...], 0.0)

    pos = tgt > 0
    posf = pos.astype(jnp.float32)
    ad1 = jnp.abs(d1)
    sl1 = jnp.where(ad1 < 1.0, 0.5 * d1 * d1, ad1 - 0.5)
    ad2 = jnp.abs(d2)
    sl2 = jnp.where(ad2 < 1.0, 0.5 * d2 * d2, ad2 - 0.5)
    l1_sum = jnp.sum(sl1 * posf)
    l2_sum = jnp.sum(sl2 * posf)
    reg_num = jnp.sum(posf)

    x0 = osp[:, 0:1]
    x1 = osp[:, 1:2]
    m = jnp.maximum(x0, x1)
    e0 = jnp.exp(x0 - m)
    e1 = jnp.exp(x1 - m)
    se = e0 + e1
    lse = m + jnp.log(se)
    p1 = e1 / se
    xy = jnp.where(ost == 1, x1, x0)
    logpt = xy - lse
    pt = jnp.exp(logpt)
    alpha_t = jnp.where(ost == 0, 1.0 - _ALPHA, _ALPHA)
    focal = -alpha_t * logpt * (1.0 - pt) * (1.0 - pt)
    focal_sum = jnp.sum(focal * maskf)
    pos_num = jnp.sum(jnp.where(ost > 0, maskf, 0.0))

    cm = jnp.max(conf, axis=1, keepdims=True)
    ce_exp = jnp.exp(conf - cm)
    cse = jnp.sum(ce_exp, axis=1, keepdims=True)
    clse = cm + jnp.log(cse)
    lanes = jax.lax.broadcasted_iota(jnp.int32, (ROWS, C), 1)
    xt = jnp.sum(jnp.where(lanes == tgt, conf, 0.0), axis=1, keepdims=True)
    ce = clse - xt
    os_pos = p1 > _OBJ_THRESH
    sel = jnp.where(jnp.logical_or(pos, os_pos), maskf, 0.0)
    ce_sum = jnp.sum(ce * sel)
    sel_sum = jnp.sum(sel)

    out_ref[0] += l1_sum
    out_ref[1] += l2_sum
    out_ref[2] += reg_num
    out_ref[3] += focal_sum
    out_ref[4] += pos_num
    out_ref[5] += ce_sum
    out_ref[6] += sel_sum

    @pl.when(pid == GRID - 1)
    def _finalize():
        l1 = out_ref[0]
        l2 = out_ref[1]
        rn = out_ref[2]
        fs = out_ref[3]
        pn = out_ref[4]
        cs = out_ref[5]
        ss = out_ref[6]
        loc_loss = (l2 * 0.5 + l1 * 0.35) / jnp.maximum(rn, 1.0)
        os_loss = fs * 10.0
        os_loss = jnp.where(pn > 0, os_loss / jnp.maximum(pn, 1.0),
                            os_loss / 500.0)
        conf_loss = cs / jnp.maximum(ss, 1.0)
        out_ref[0] = loc_loss
        out_ref[1] = os_loss
        out_ref[2] = conf_loss


@functools.partial(jax.jit, static_argnames=("interpret",))
def _fused(loc1, loc2, loct, conf, tgt, osp, ost, interpret=False):
    conf2 = conf.reshape(TOTAL, C)
    loc1r = loc1.reshape(TOTAL, 4)
    loc2r = loc2.reshape(TOTAL, 4)
    loctr = loct.reshape(TOTAL, 4)
    tgt2 = tgt.reshape(TOTAL, 1)
    osp2 = osp.reshape(TOTAL, 2)
    ost2 = ost.reshape(TOTAL, 1)

    def rowblock(width):
        return pl.BlockSpec((ROWS, width), lambda i: (i, 0))

    out = pl.pallas_call(
        _body,
        grid=(GRID,),
        in_specs=[
            rowblock(C),
            rowblock(1),
            rowblock(2),
            rowblock(1),
            rowblock(4),
            rowblock(4),
            rowblock(4),
        ],
        out_specs=pl.BlockSpec(memory_space=pltpu.SMEM),
        out_shape=jax.ShapeDtypeStruct((8,), jnp.float32),
        compiler_params=pltpu.CompilerParams(
            dimension_semantics=("arbitrary",),
        ),
        interpret=interpret,
    )(conf2, tgt2, osp2, ost2, loc1r, loc2r, loctr)
    return out[0], out[1], out[2]


def kernel(loc1_preds, loc2_preds, loc_targets, conf_preds, conf_targets,
           os_pred, os_target):
    return _fused(loc1_preds, loc2_preds, loc_targets, conf_preds,
                  conf_targets, os_pred, os_target)
